# baseline (speedup 1.0000x reference)
import functools

import jax
import jax.numpy as jnp
from jax import lax
from jax.experimental import pallas as pl
from jax.experimental.pallas import tpu as pltpu
from jax.experimental.pallas import tpu_sc as plsc

NUM_NODES = 10000
NUM_EDGES = 320000
NUM_GRAPHS = 64
HID = 256

NSUB = 16
NCORE = 2
STRIPE = 624
TAIL_BASE = NSUB * STRIPE
TAIL = NUM_NODES - TAIL_BASE

CS_CHUNK = 50
CS_BLK = 16
CS_NBLK = NUM_EDGES // NSUB // (CS_CHUNK * CS_BLK)
ES_CHUNK = 50
ES_BLK = 8
ES_NBLK = NUM_EDGES // (NCORE * NSUB) // (ES_CHUNK * ES_BLK)


def _segsum_body(chan_split, chunk, blk, nblk, h_hbm, src_hbm, dst_hbm,
                 zeros_hbm, out_hbm, src_set, dst_set, rows, accum,
                 gsem, ssem, isem):
    c = lax.axis_index("c")
    s = lax.axis_index("s")
    if chan_split:
        h_c = h_hbm.at[c]
        widx = s
    else:
        h_c = h_hbm
        widx = c * NSUB + s

    def idx_pair(b, p):
        return (pltpu.make_async_copy(src_hbm.at[widx, b], src_set.at[p],
                                      isem.at[p]),
                pltpu.make_async_copy(dst_hbm.at[widx, b], dst_set.at[p],
                                      isem.at[p]))

    def g_copy(p, k, m):
        return pltpu.make_async_copy(h_c.at[src_set.at[p, k]], rows.at[m],
                                     gsem.at[m])

    def s_start(p, k, m):
        pltpu.async_copy(rows.at[m], accum.at[dst_set.at[p, k]], ssem.at[m],
                         add=True)

    def s_wait(p, k, m):
        pltpu.make_async_copy(rows.at[m], accum.at[dst_set.at[p, k]],
                              ssem.at[m]).wait()

    d1, d2 = idx_pair(0, 0)
    d1.start()
    d2.start()
    pltpu.sync_copy(zeros_hbm.at[pl.ds(s * STRIPE, STRIPE)],
                    accum.at[pl.ds(s * STRIPE, STRIPE)])

    @pl.when(s == NSUB - 1)
    def _():
        pltpu.sync_copy(zeros_hbm.at[pl.ds(TAIL_BASE, TAIL)],
                        accum.at[pl.ds(TAIL_BASE, TAIL)])

    plsc.subcore_barrier()
    d1.wait()
    d2.wait()
    g_copy(0, 0, 0).start()
    g_copy(0, 1, 1).start()

    def block_step(b, carry):
        p = b % 2
        q = 1 - p
        for k in range(blk):
            m = k % 4
            g_copy(p, k, m).wait()
            s_start(p, k, m)
            if k >= 2:
                s_wait(p, k - 2, (k - 2) % 4)
            else:
                @pl.when(b > 0)
                def _():
                    s_wait(q, blk - 2 + k, (k - 2) % 4)
            if k == 1:
                @pl.when(b + 1 < nblk)
                def _():
                    e1, e2 = idx_pair(b + 1, q)
                    e1.start()
                    e2.start()
            if k < blk - 2:
                g_copy(p, k + 2, (k + 2) % 4).start()
            else:
                if k == blk - 2:
                    @pl.when(b + 1 < nblk)
                    def _():
                        e1, e2 = idx_pair(b + 1, q)
                        e1.wait()
                        e2.wait()

                @pl.when(b + 1 < nblk)
                def _():
                    g_copy(q, k + 2 - blk, (k + 2) % 4).start()
        return carry

    lax.fori_loop(0, nblk, block_step, 0)
    pe = (nblk - 1) % 2
    s_wait(pe, blk - 2, (blk - 2) % 4)
    s_wait(pe, blk - 1, (blk - 1) % 4)
    plsc.subcore_barrier()
    pltpu.sync_copy(accum.at[pl.ds(s * STRIPE, STRIPE)],
                    out_hbm.at[c].at[pl.ds(s * STRIPE, STRIPE)])

    @pl.when(s == NSUB - 1)
    def _():
        pltpu.sync_copy(accum.at[pl.ds(TAIL_BASE, TAIL)],
                        out_hbm.at[c].at[pl.ds(TAIL_BASE, TAIL)])


def _segsum_sc(chan_split, h_arr, src_r, dst_r, zeros):
    chunk = CS_CHUNK if chan_split else ES_CHUNK
    blk = CS_BLK if chan_split else ES_BLK
    nblk = CS_NBLK if chan_split else ES_NBLK
    mesh = plsc.VectorSubcoreMesh(core_axis_name="c", subcore_axis_name="s")
    return pl.kernel(
        functools.partial(_segsum_body, chan_split, chunk, blk, nblk),
        out_type=jax.ShapeDtypeStruct((NCORE, NUM_NODES, 128), jnp.float32),
        mesh=mesh,
        scratch_types=[
            pltpu.VMEM((2, blk, chunk), jnp.int32),
            pltpu.VMEM((2, blk, chunk), jnp.int32),
            pltpu.VMEM((4, chunk, 128), jnp.float32),
            pltpu.VMEM_SHARED((NUM_NODES, 128), jnp.float32),
            pltpu.SemaphoreType.DMA((4,)),
            pltpu.SemaphoreType.DMA((4,)),
            pltpu.SemaphoreType.DMA((2,)),
        ],
        name="segsum_sc",
    )(h_arr, src_r, dst_r, zeros)


def _bn_cols(t, gamma, beta, eps=1e-5):
    mean = jnp.mean(t, axis=0, keepdims=True)
    var = jnp.mean((t - mean) ** 2, axis=0, keepdims=True)
    return gamma * (t - mean) * lax.rsqrt(var + eps) + beta


def _layer_body(first, h_ref, agg_ref, eps_ref, w1_ref, b1_ref, g1_ref,
                be1_ref, w2_ref, b2_ref, g2_ref, be2_ref, out_ref):
    eps = eps_ref[0, 0]
    if first:
        h = h_ref[...]
        agg = agg_ref[0] + agg_ref[1]
    else:
        h = jnp.concatenate([h_ref[0], h_ref[1]], axis=1)
        agg = jnp.concatenate([agg_ref[0], agg_ref[1]], axis=1)
    z = (1.0 + eps) * h + agg
    t = jnp.dot(z, w1_ref[...], preferred_element_type=jnp.float32) + b1_ref[...]
    t = _bn_cols(t, g1_ref[...], be1_ref[...])
    t = jnp.maximum(t, 0.0)
    u = jnp.dot(t, w2_ref[...], preferred_element_type=jnp.float32) + b2_ref[...]
    u = _bn_cols(u, g2_ref[...], be2_ref[...])
    u = jnp.maximum(u, 0.0)
    out_ref[0] = u[:, :HID // 2]
    out_ref[1] = u[:, HID // 2:]


def _gin_layer(first, h_arr, agg_arr, p):
    eps2d = p['eps'].reshape(1, 1)
    return pl.pallas_call(
        functools.partial(_layer_body, first),
        out_shape=jax.ShapeDtypeStruct((2, NUM_NODES, HID // 2), jnp.float32),
    )(h_arr, agg_arr,
      eps2d, p['W1'], p['b1'].reshape(1, -1), p['bn_g'].reshape(1, -1),
      p['bn_b'].reshape(1, -1), p['W2'], p['b2'].reshape(1, -1),
      p['obn_g'].reshape(1, -1), p['obn_b'].reshape(1, -1))


def _head_body(h_ref, batch_ref, w1_ref, b1_ref, g1_ref, be1_ref,
               w2_ref, b2_ref, out_ref):
    h = jnp.concatenate([h_ref[0], h_ref[1]], axis=1)
    gids = lax.broadcasted_iota(jnp.int32, (NUM_GRAPHS, NUM_NODES), 0)
    onehot = (batch_ref[...] == gids).astype(jnp.float32)
    g = jnp.dot(onehot, h, preferred_element_type=jnp.float32)
    g = jnp.dot(g, w1_ref[...], preferred_element_type=jnp.float32) + b1_ref[...]
    g = _bn_cols(g, g1_ref[...], be1_ref[...])
    g = jnp.maximum(g, 0.0)
    g = jnp.dot(g, w2_ref[...], preferred_element_type=jnp.float32) + b2_ref[...]
    m = jnp.max(g, axis=1, keepdims=True)
    e = g - m
    lse = jnp.log(jnp.sum(jnp.exp(e), axis=1, keepdims=True))
    out_ref[...] = e - lse


def _head(h_split, batch, params):
    return pl.pallas_call(
        _head_body,
        out_shape=jax.ShapeDtypeStruct((NUM_GRAPHS, params['lin2_W'].shape[1]),
                                       jnp.float32),
    )(h_split, batch.reshape(1, NUM_NODES),
      params['lin1_W'], params['lin1_b'].reshape(1, -1),
      params['bn1_g'].reshape(1, -1), params['bn1_b'].reshape(1, -1),
      params['lin2_W'], params['lin2_b'].reshape(1, -1))


def kernel(x, edge_index, batch, params):
    src_es = edge_index[0].reshape(NCORE * NSUB, ES_NBLK, ES_BLK, ES_CHUNK)
    dst_es = edge_index[1].reshape(NCORE * NSUB, ES_NBLK, ES_BLK, ES_CHUNK)
    src_cs = edge_index[0].reshape(NSUB, CS_NBLK, CS_BLK, CS_CHUNK)
    dst_cs = edge_index[1].reshape(NSUB, CS_NBLK, CS_BLK, CS_CHUNK)
    zeros = jnp.zeros((NUM_NODES, 128), jnp.float32)

    agg2 = _segsum_sc(False, x, src_es, dst_es, zeros)
    h_split = _gin_layer(True, x, agg2, params['conv0'])

    for i in (1, 2):
        agg_split = _segsum_sc(True, h_split, src_cs, dst_cs, zeros)
        h_split = _gin_layer(False, h_split, agg_split, params['conv%d' % i])

    return _head(h_split, batch, params)


# speedup vs baseline: 1.2800x; 1.2800x over previous
import functools

import jax
import jax.numpy as jnp
from jax import lax
from jax.experimental import pallas as pl
from jax.experimental.pallas import tpu as pltpu
from jax.experimental.pallas import tpu_sc as plsc

NUM_NODES = 10000
NUM_EDGES = 320000
NUM_GRAPHS = 64
HID = 256

NSUB = 16
NCORE = 2
STRIPE = 624
TAIL_BASE = NSUB * STRIPE
TAIL = NUM_NODES - TAIL_BASE

CS_CHUNK = 100
CS_BLK = 8
CS_NBLK = NUM_EDGES // NSUB // (CS_CHUNK * CS_BLK)
ES_CHUNK = 100
ES_BLK = 5
ES_NBLK = NUM_EDGES // (NCORE * NSUB) // (ES_CHUNK * ES_BLK)


def _segsum_body(chan_split, chunk, blk, nblk, h_hbm, src_hbm, dst_hbm,
                 zeros_hbm, out_hbm, src_set, dst_set, rows, accum,
                 gsem, ssem, isem):
    c = lax.axis_index("c")
    s = lax.axis_index("s")
    if chan_split:
        h_c = h_hbm.at[c]
        widx = s
    else:
        h_c = h_hbm
        widx = c * NSUB + s

    def idx_pair(b, p):
        return (pltpu.make_async_copy(src_hbm.at[widx, b], src_set.at[p],
                                      isem.at[p]),
                pltpu.make_async_copy(dst_hbm.at[widx, b], dst_set.at[p],
                                      isem.at[p]))

    def g_copy(p, k, m):
        return pltpu.make_async_copy(h_c.at[src_set.at[p, k]], rows.at[m],
                                     gsem.at[m])

    def s_start(p, k, m):
        pltpu.async_copy(rows.at[m], accum.at[dst_set.at[p, k]], ssem.at[m],
                         add=True)

    def s_wait(p, k, m):
        pltpu.make_async_copy(rows.at[m], accum.at[dst_set.at[p, k]],
                              ssem.at[m]).wait()

    d1, d2 = idx_pair(0, 0)
    d1.start()
    d2.start()
    pltpu.sync_copy(zeros_hbm.at[pl.ds(s * STRIPE, STRIPE)],
                    accum.at[pl.ds(s * STRIPE, STRIPE)])

    @pl.when(s == NSUB - 1)
    def _():
        pltpu.sync_copy(zeros_hbm.at[pl.ds(TAIL_BASE, TAIL)],
                        accum.at[pl.ds(TAIL_BASE, TAIL)])

    plsc.subcore_barrier()
    d1.wait()
    d2.wait()
    g_copy(0, 0, 0).start()
    g_copy(0, 1, 1).start()

    total = nblk * blk

    def pkm(g):
        return (g // blk) % 2, g % blk, g % 3

    def chunk_step(g, carry):
        b = g // blk
        p, k, m = pkm(g)
        q = 1 - p
        g_copy(p, k, m).wait()
        s_start(p, k, m)

        @pl.when(g >= 1)
        def _():
            s_wait(*pkm(g - 1))

        @pl.when(jnp.logical_and(k == 0, b + 1 < nblk))
        def _():
            e1, e2 = idx_pair(b + 1, q)
            e1.start()
            e2.start()

        @pl.when(jnp.logical_and(k == blk - 2, b + 1 < nblk))
        def _():
            e1, e2 = idx_pair(b + 1, q)
            e1.wait()
            e2.wait()

        @pl.when(g + 2 < total)
        def _():
            g_copy(*pkm(g + 2)).start()

        return carry

    lax.fori_loop(0, total, chunk_step, 0)
    s_wait(*pkm(total - 1))
    plsc.subcore_barrier()
    pltpu.sync_copy(accum.at[pl.ds(s * STRIPE, STRIPE)],
                    out_hbm.at[c].at[pl.ds(s * STRIPE, STRIPE)])

    @pl.when(s == NSUB - 1)
    def _():
        pltpu.sync_copy(accum.at[pl.ds(TAIL_BASE, TAIL)],
                        out_hbm.at[c].at[pl.ds(TAIL_BASE, TAIL)])


def _segsum_sc(chan_split, h_arr, src_r, dst_r, zeros):
    chunk = CS_CHUNK if chan_split else ES_CHUNK
    blk = CS_BLK if chan_split else ES_BLK
    nblk = CS_NBLK if chan_split else ES_NBLK
    mesh = plsc.VectorSubcoreMesh(core_axis_name="c", subcore_axis_name="s")
    return pl.kernel(
        functools.partial(_segsum_body, chan_split, chunk, blk, nblk),
        out_type=jax.ShapeDtypeStruct((NCORE, NUM_NODES, 128), jnp.float32),
        mesh=mesh,
        scratch_types=[
            pltpu.VMEM((2, blk, chunk), jnp.int32),
            pltpu.VMEM((2, blk, chunk), jnp.int32),
            pltpu.VMEM((3, chunk, 128), jnp.float32),
            pltpu.VMEM_SHARED((NUM_NODES, 128), jnp.float32),
            pltpu.SemaphoreType.DMA((3,)),
            pltpu.SemaphoreType.DMA((3,)),
            pltpu.SemaphoreType.DMA((2,)),
        ],
        name="segsum_sc",
    )(h_arr, src_r, dst_r, zeros)


def _bn_cols(t, gamma, beta, eps=1e-5):
    mean = jnp.mean(t, axis=0, keepdims=True)
    var = jnp.mean((t - mean) ** 2, axis=0, keepdims=True)
    return gamma * (t - mean) * lax.rsqrt(var + eps) + beta


def _layer_body(first, h_ref, agg_ref, eps_ref, w1_ref, b1_ref, g1_ref,
                be1_ref, w2_ref, b2_ref, g2_ref, be2_ref, out_ref):
    eps = eps_ref[0, 0]
    if first:
        h = h_ref[...]
        agg = agg_ref[0] + agg_ref[1]
    else:
        h = jnp.concatenate([h_ref[0], h_ref[1]], axis=1)
        agg = jnp.concatenate([agg_ref[0], agg_ref[1]], axis=1)
    z = (1.0 + eps) * h + agg
    t = jnp.dot(z, w1_ref[...], preferred_element_type=jnp.float32) + b1_ref[...]
    t = _bn_cols(t, g1_ref[...], be1_ref[...])
    t = jnp.maximum(t, 0.0)
    u = jnp.dot(t, w2_ref[...], preferred_element_type=jnp.float32) + b2_ref[...]
    u = _bn_cols(u, g2_ref[...], be2_ref[...])
    u = jnp.maximum(u, 0.0)
    out_ref[0] = u[:, :HID // 2]
    out_ref[1] = u[:, HID // 2:]


def _gin_layer(first, h_arr, agg_arr, p):
    eps2d = p['eps'].reshape(1, 1)
    return pl.pallas_call(
        functools.partial(_layer_body, first),
        out_shape=jax.ShapeDtypeStruct((2, NUM_NODES, HID // 2), jnp.float32),
    )(h_arr, agg_arr,
      eps2d, p['W1'], p['b1'].reshape(1, -1), p['bn_g'].reshape(1, -1),
      p['bn_b'].reshape(1, -1), p['W2'], p['b2'].reshape(1, -1),
      p['obn_g'].reshape(1, -1), p['obn_b'].reshape(1, -1))


def _head_body(h_ref, batch_ref, w1_ref, b1_ref, g1_ref, be1_ref,
               w2_ref, b2_ref, out_ref):
    h = jnp.concatenate([h_ref[0], h_ref[1]], axis=1)
    gids = lax.broadcasted_iota(jnp.int32, (NUM_GRAPHS, NUM_NODES), 0)
    onehot = (batch_ref[...] == gids).astype(jnp.float32)
    g = jnp.dot(onehot, h, preferred_element_type=jnp.float32)
    g = jnp.dot(g, w1_ref[...], preferred_element_type=jnp.float32) + b1_ref[...]
    g = _bn_cols(g, g1_ref[...], be1_ref[...])
    g = jnp.maximum(g, 0.0)
    g = jnp.dot(g, w2_ref[...], preferred_element_type=jnp.float32) + b2_ref[...]
    m = jnp.max(g, axis=1, keepdims=True)
    e = g - m
    lse = jnp.log(jnp.sum(jnp.exp(e), axis=1, keepdims=True))
    out_ref[...] = e - lse


def _head(h_split, batch, params):
    return pl.pallas_call(
        _head_body,
        out_shape=jax.ShapeDtypeStruct((NUM_GRAPHS, params['lin2_W'].shape[1]),
                                       jnp.float32),
    )(h_split, batch.reshape(1, NUM_NODES),
      params['lin1_W'], params['lin1_b'].reshape(1, -1),
      params['bn1_g'].reshape(1, -1), params['bn1_b'].reshape(1, -1),
      params['lin2_W'], params['lin2_b'].reshape(1, -1))


def kernel(x, edge_index, batch, params):
    src_es = edge_index[0].reshape(NCORE * NSUB, ES_NBLK, ES_BLK, ES_CHUNK)
    dst_es = edge_index[1].reshape(NCORE * NSUB, ES_NBLK, ES_BLK, ES_CHUNK)
    src_cs = edge_index[0].reshape(NSUB, CS_NBLK, CS_BLK, CS_CHUNK)
    dst_cs = edge_index[1].reshape(NSUB, CS_NBLK, CS_BLK, CS_CHUNK)
    zeros = jnp.zeros((NUM_NODES, 128), jnp.float32)

    agg2 = _segsum_sc(False, x, src_es, dst_es, zeros)
    h_split = _gin_layer(True, x, agg2, params['conv0'])

    for i in (1, 2):
        agg_split = _segsum_sc(True, h_split, src_cs, dst_cs, zeros)
        h_split = _gin_layer(False, h_split, agg_split, params['conv%d' % i])

    return _head(h_split, batch, params)


# speedup vs baseline: 1.2883x; 1.0065x over previous
import functools

import jax
import jax.numpy as jnp
from jax import lax
from jax.experimental import pallas as pl
from jax.experimental.pallas import tpu as pltpu
from jax.experimental.pallas import tpu_sc as plsc

NUM_NODES = 10000
NUM_EDGES = 320000
NUM_GRAPHS = 64
HID = 256

NSUB = 16
NCORE = 2
STRIPE = 624
TAIL_BASE = NSUB * STRIPE
TAIL = NUM_NODES - TAIL_BASE

CS_CHUNK = 100
CS_BLK = 8
CS_NBLK = NUM_EDGES // NSUB // (CS_CHUNK * CS_BLK)
ES_CHUNK = 100
ES_BLK = 5
ES_NBLK = NUM_EDGES // (NCORE * NSUB) // (ES_CHUNK * ES_BLK)


def _segsum_body(chan_split, chunk, blk, nblk, h_hbm, src_hbm, dst_hbm,
                 zeros_hbm, out_hbm, src_set, dst_set, rows, accum,
                 gsem, ssem, isem):
    c = lax.axis_index("c")
    s = lax.axis_index("s")
    if chan_split:
        h_c = h_hbm.at[c]
        widx = s
    else:
        h_c = h_hbm
        widx = c * NSUB + s

    def idx_pair(b, p):
        return (pltpu.make_async_copy(src_hbm.at[widx, b], src_set.at[p],
                                      isem.at[p]),
                pltpu.make_async_copy(dst_hbm.at[widx, b], dst_set.at[p],
                                      isem.at[p]))

    def g_copy(p, k, m):
        return pltpu.make_async_copy(h_c.at[src_set.at[p, k]], rows.at[m],
                                     gsem.at[m])

    def s_start(p, k, m):
        pltpu.async_copy(rows.at[m], accum.at[dst_set.at[p, k]], ssem.at[m],
                         add=True)

    def s_wait(p, k, m):
        pltpu.make_async_copy(rows.at[m], accum.at[dst_set.at[p, k]],
                              ssem.at[m]).wait()

    d1, d2 = idx_pair(0, 0)
    d1.start()
    d2.start()
    pltpu.sync_copy(zeros_hbm.at[pl.ds(s * STRIPE, STRIPE)],
                    accum.at[pl.ds(s * STRIPE, STRIPE)])

    @pl.when(s == NSUB - 1)
    def _():
        pltpu.sync_copy(zeros_hbm.at[pl.ds(TAIL_BASE, TAIL)],
                        accum.at[pl.ds(TAIL_BASE, TAIL)])

    plsc.subcore_barrier()
    d1.wait()
    d2.wait()
    g_copy(0, 0, 0).start()
    g_copy(0, 1, 1).start()

    total = nblk * blk

    def pkm(g):
        return (g // blk) % 2, g % blk, g % 3

    def chunk_step(g, carry):
        b = g // blk
        p, k, m = pkm(g)
        q = 1 - p
        g_copy(p, k, m).wait()
        s_start(p, k, m)

        @pl.when(g >= 1)
        def _():
            s_wait(*pkm(g - 1))

        @pl.when(jnp.logical_and(k == 0, b + 1 < nblk))
        def _():
            e1, e2 = idx_pair(b + 1, q)
            e1.start()
            e2.start()

        @pl.when(jnp.logical_and(k == blk - 2, b + 1 < nblk))
        def _():
            e1, e2 = idx_pair(b + 1, q)
            e1.wait()
            e2.wait()

        @pl.when(g + 2 < total)
        def _():
            g_copy(*pkm(g + 2)).start()

        return carry

    lax.fori_loop(0, total, chunk_step, 0)
    s_wait(*pkm(total - 1))
    plsc.subcore_barrier()
    pltpu.sync_copy(accum.at[pl.ds(s * STRIPE, STRIPE)],
                    out_hbm.at[c].at[pl.ds(s * STRIPE, STRIPE)])

    @pl.when(s == NSUB - 1)
    def _():
        pltpu.sync_copy(accum.at[pl.ds(TAIL_BASE, TAIL)],
                        out_hbm.at[c].at[pl.ds(TAIL_BASE, TAIL)])


def _segsum_sc(chan_split, h_arr, src_r, dst_r, zeros):
    chunk = CS_CHUNK if chan_split else ES_CHUNK
    blk = CS_BLK if chan_split else ES_BLK
    nblk = CS_NBLK if chan_split else ES_NBLK
    mesh = plsc.VectorSubcoreMesh(core_axis_name="c", subcore_axis_name="s")
    return pl.kernel(
        functools.partial(_segsum_body, chan_split, chunk, blk, nblk),
        out_type=jax.ShapeDtypeStruct((NCORE, NUM_NODES, 128), jnp.float32),
        mesh=mesh,
        scratch_types=[
            pltpu.VMEM((2, blk, chunk), jnp.int32),
            pltpu.VMEM((2, blk, chunk), jnp.int32),
            pltpu.VMEM((3, chunk, 128), jnp.float32),
            pltpu.VMEM_SHARED((NUM_NODES, 128), jnp.float32),
            pltpu.SemaphoreType.DMA((3,)),
            pltpu.SemaphoreType.DMA((3,)),
            pltpu.SemaphoreType.DMA((2,)),
        ],
        name="segsum_sc",
    )(h_arr, src_r, dst_r, zeros)


def _bn_cols(t, gamma, beta, eps=1e-5):
    mean = jnp.mean(t, axis=0, keepdims=True)
    var = jnp.mean((t - mean) ** 2, axis=0, keepdims=True)
    return gamma * (t - mean) * lax.rsqrt(var + eps) + beta


def _gin_mlp(first, h_ref, agg_ref, eps_ref, w1a_ref, w1b_ref, b1_ref,
             g1_ref, be1_ref, w2_ref, b2_ref, g2_ref, be2_ref):
    eps = eps_ref[0, 0]
    if first:
        z = (1.0 + eps) * h_ref[...] + agg_ref[0] + agg_ref[1]
        t = jnp.dot(z, w1a_ref[...], preferred_element_type=jnp.float32)
    else:
        z0 = (1.0 + eps) * h_ref[0] + agg_ref[0]
        z1 = (1.0 + eps) * h_ref[1] + agg_ref[1]
        t = (jnp.dot(z0, w1a_ref[...], preferred_element_type=jnp.float32)
             + jnp.dot(z1, w1b_ref[...], preferred_element_type=jnp.float32))
    t = t + b1_ref[...]
    t = _bn_cols(t, g1_ref[...], be1_ref[...])
    t = jnp.maximum(t, 0.0)
    u = jnp.dot(t, w2_ref[...], preferred_element_type=jnp.float32) + b2_ref[...]
    u = _bn_cols(u, g2_ref[...], be2_ref[...])
    return jnp.maximum(u, 0.0)


def _layer_body(first, *refs):
    out_ref = refs[-1]
    u = _gin_mlp(first, *refs[:-1])
    out_ref[0] = u[:, :HID // 2]
    out_ref[1] = u[:, HID // 2:]


def _layer_args(p, first):
    eps2d = p['eps'].reshape(1, 1)
    if first:
        w1a, w1b = p['W1'], p['W1'][:1]
    else:
        w1a, w1b = p['W1'][:HID // 2], p['W1'][HID // 2:]
    return (eps2d, w1a, w1b, p['b1'].reshape(1, -1), p['bn_g'].reshape(1, -1),
            p['bn_b'].reshape(1, -1), p['W2'], p['b2'].reshape(1, -1),
            p['obn_g'].reshape(1, -1), p['obn_b'].reshape(1, -1))


def _gin_layer(first, h_arr, agg_arr, p):
    return pl.pallas_call(
        functools.partial(_layer_body, first),
        out_shape=jax.ShapeDtypeStruct((2, NUM_NODES, HID // 2), jnp.float32),
    )(h_arr, agg_arr, *_layer_args(p, first))


def _final_body(h_ref, agg_ref, eps_ref, w1a_ref, w1b_ref, b1_ref, g1_ref,
                be1_ref, w2_ref, b2_ref, g2_ref, be2_ref, batch_ref,
                l1w_ref, l1b_ref, bg_ref, bb_ref, l2w_ref, l2b_ref, out_ref):
    u = _gin_mlp(False, h_ref, agg_ref, eps_ref, w1a_ref, w1b_ref, b1_ref,
                 g1_ref, be1_ref, w2_ref, b2_ref, g2_ref, be2_ref)
    gids = lax.broadcasted_iota(jnp.int32, (NUM_GRAPHS, NUM_NODES), 0)
    onehot = (batch_ref[...] == gids).astype(jnp.float32)
    g = jnp.dot(onehot, u, preferred_element_type=jnp.float32)
    g = jnp.dot(g, l1w_ref[...], preferred_element_type=jnp.float32) + l1b_ref[...]
    g = _bn_cols(g, bg_ref[...], bb_ref[...])
    g = jnp.maximum(g, 0.0)
    g = jnp.dot(g, l2w_ref[...], preferred_element_type=jnp.float32) + l2b_ref[...]
    m = jnp.max(g, axis=1, keepdims=True)
    e = g - m
    lse = jnp.log(jnp.sum(jnp.exp(e), axis=1, keepdims=True))
    out_ref[...] = e - lse


def _final_layer(h_arr, agg_arr, p, batch, params):
    return pl.pallas_call(
        _final_body,
        out_shape=jax.ShapeDtypeStruct((NUM_GRAPHS, params['lin2_W'].shape[1]),
                                       jnp.float32),
    )(h_arr, agg_arr, *_layer_args(p, False), batch.reshape(1, NUM_NODES),
      params['lin1_W'], params['lin1_b'].reshape(1, -1),
      params['bn1_g'].reshape(1, -1), params['bn1_b'].reshape(1, -1),
      params['lin2_W'], params['lin2_b'].reshape(1, -1))


def kernel(x, edge_index, batch, params):
    src_es = edge_index[0].reshape(NCORE * NSUB, ES_NBLK, ES_BLK, ES_CHUNK)
    dst_es = edge_index[1].reshape(NCORE * NSUB, ES_NBLK, ES_BLK, ES_CHUNK)
    src_cs = edge_index[0].reshape(NSUB, CS_NBLK, CS_BLK, CS_CHUNK)
    dst_cs = edge_index[1].reshape(NSUB, CS_NBLK, CS_BLK, CS_CHUNK)
    zeros = jnp.zeros((NUM_NODES, 128), jnp.float32)

    agg2 = _segsum_sc(False, x, src_es, dst_es, zeros)
    h_split = _gin_layer(True, x, agg2, params['conv0'])

    agg_split = _segsum_sc(True, h_split, src_cs, dst_cs, zeros)
    h_split = _gin_layer(False, h_split, agg_split, params['conv1'])

    agg_split = _segsum_sc(True, h_split, src_cs, dst_cs, zeros)
    return _final_layer(h_split, agg_split, params['conv2'], batch, params)
